# baseline (speedup 1.0000x reference)
import functools

import jax
import jax.numpy as jnp
from jax import lax
from jax.experimental import pallas as pl
from jax.experimental.pallas import tpu as pltpu
from jax.experimental.pallas import tpu_sc as plsc

N = 20000
NP = 20480
NS1 = NP // 16
NS2 = NS1 // 16
KPAD = 112
MAX_DET = 100
SCORE_THRESH = 0.05
NMS_THRESH = 0.5
NEG_INF = float("-inf")


def _nms_body(cat_hbm, sc_hbm, out_hbm, cat_v, work_v, seg1_v, seg2_v,
              kcat_v, out_v, sem):
    wid = lax.axis_index("s")

    @pl.when(wid == 0)
    def _():
        iota = lax.broadcasted_iota(jnp.int32, (16,), 0)
        iota_f = iota.astype(jnp.float32)
        iota16 = iota * 16
        lane0 = iota == 0
        gmod = jnp.minimum(iota, 3)
        neg16 = jnp.full((16,), NEG_INF, jnp.float32)

        cdma = pltpu.async_copy(cat_hbm, cat_v.at[pl.ds(0, 4 * N)], sem)
        pltpu.sync_copy(sc_hbm, work_v.at[pl.ds(0, N)])
        for c in range(N // 16, NP // 16):
            work_v[pl.ds(c * 16, 16)] = jnp.full((16,), -1.0, jnp.float32)

        def s1(c, _):
            m = neg16
            for j in range(16):
                s = plsc.load_gather(work_v, [c * 256 + iota16 + j])
                m = jnp.maximum(m, jnp.where(s >= SCORE_THRESH, s, NEG_INF))
            seg1_v[pl.ds(c * 16, 16)] = m
            return 0
        lax.fori_loop(0, NS2, s1, 0, unroll=False)

        def s2(c, _):
            m = neg16
            for j in range(16):
                m = jnp.maximum(m, plsc.load_gather(seg1_v,
                                                    [c * 256 + iota16 + j]))
            seg2_v[pl.ds(c * 16, 16)] = m
            return 0
        lax.fori_loop(0, NS2 // 16, s2, 0, unroll=True)

        for p, val in enumerate((2.0, 2.0, -2.0, -2.0, 0.0)):
            for c in range(KPAD // 16):
                kcat_v[pl.ds(p * KPAD + c * 16, 16)] = jnp.full(
                    (16,), val, jnp.float32)

        cdma.wait()

        def pick():
            def seg_scan(c, carry):
                m, mi = carry
                v = seg2_v[pl.ds(c * 16, 16)]
                upd = v > m
                return jnp.where(upd, v, m), jnp.where(upd, c * 16.0 + iota_f,
                                                       mi)
            m, mi = lax.fori_loop(0, NS2 // 16, seg_scan, (neg16, iota_f),
                                  unroll=True)
            vmax = jnp.max(m)
            j2 = jnp.min(jnp.where(m == vmax, mi, jnp.float32(1e9)))
            j2 = j2.astype(jnp.int32)
            c1 = seg1_v[pl.ds(j2 * 16, 16)]
            j1 = j2 * 16 + jnp.minimum(plsc.all_reduce_ffs(c1 == vmax)[0], 15)
            c0 = work_v[pl.ds(j1 * 16, 16)]
            g = j1 * 16 + jnp.minimum(plsc.all_reduce_ffs(c0 == vmax)[0], 15)
            return vmax, g

        def kept_test(it, vmax, g):
            gc = plsc.load_gather(cat_v, [g * 4 + gmod])
            cx1, cy1, cx2, cy2 = gc[0], gc[1], gc[2], gc[3]
            carea = jnp.maximum(cx2 - cx1, 0.0) * jnp.maximum(cy2 - cy1, 0.0)
            nch = (it + 15) // 16

            def tb(k, acc):
                kx1 = kcat_v[pl.ds(k * 16, 16)]
                ky1 = kcat_v[pl.ds(KPAD + k * 16, 16)]
                kx2 = kcat_v[pl.ds(2 * KPAD + k * 16, 16)]
                ky2 = kcat_v[pl.ds(3 * KPAD + k * 16, 16)]
                karea = kcat_v[pl.ds(4 * KPAD + k * 16, 16)]
                x1 = jnp.maximum(kx1, cx1)
                y1 = jnp.maximum(ky1, cy1)
                x2 = jnp.minimum(kx2, cx2)
                y2 = jnp.minimum(ky2, cy2)
                inter = jnp.maximum(x2 - x1, 0.0) * jnp.maximum(y2 - y1, 0.0)
                union = karea + carea - inter
                iou = inter / jnp.maximum(union, 1e-9)
                return acc | (iou > NMS_THRESH)
            acc = lax.fori_loop(0, nch, tb, iota < 0, unroll=False)
            return jnp.any(acc) & (vmax > NEG_INF)

        def mark(g):
            plsc.store_scatter(work_v, [jnp.full((16,), g, jnp.int32)],
                               neg16, mask=lane0)
            j1 = g // 16
            c0 = work_v[pl.ds(j1 * 16, 16)]
            nm1 = jnp.max(jnp.where(c0 >= SCORE_THRESH, c0, NEG_INF))
            plsc.store_scatter(seg1_v, [jnp.full((16,), j1, jnp.int32)],
                               jnp.full((16,), nm1, jnp.float32), mask=lane0)
            j2 = j1 // 16
            nm2 = jnp.max(seg1_v[pl.ds(j2 * 16, 16)])
            plsc.store_scatter(seg2_v, [jnp.full((16,), j2, jnp.int32)],
                               jnp.full((16,), nm2, jnp.float32), mask=lane0)

        def round_body(it, _):
            vmax0, g0 = pick()
            fail0 = kept_test(it, vmax0, g0)

            def body(carry):
                _, g, _ = carry
                mark(g)
                vmax2, g2 = pick()
                return vmax2, g2, kept_test(it, vmax2, g2)
            vmax, g, _ = lax.while_loop(lambda c: c[2], body,
                                        (vmax0, g0, fail0))
            mark(g)
            valid = vmax > NEG_INF

            gc = plsc.load_gather(cat_v, [g * 4 + gmod])
            sent = jnp.where(iota < 2, 2.0, jnp.where(iota < 4, -2.0, 0.0))
            app = jnp.where(valid, gc, sent)
            ax1, ay1, ax2, ay2 = app[0], app[1], app[2], app[3]
            aarea = jnp.maximum(ax2 - ax1, 0.0) * jnp.maximum(ay2 - ay1, 0.0)
            for p, v in enumerate((ax1, ay1, ax2, ay2, aarea)):
                plsc.store_scatter(
                    kcat_v, [jnp.full((16,), p * KPAD, jnp.int32) + it],
                    jnp.full((16,), v, jnp.float32), mask=lane0)

            row = jnp.where(iota < 4, gc, jnp.where(iota == 4, vmax, 0.0))
            row = jnp.where(valid, row, jnp.zeros((16,), jnp.float32))
            out_v[pl.ds(it * 16, 16)] = row
            return 0

        lax.fori_loop(0, MAX_DET, round_body, 0, unroll=False)
        pltpu.sync_copy(out_v, out_hbm)


@jax.jit
def _nms(cat, sc):
    mesh = plsc.VectorSubcoreMesh(core_axis_name="c", subcore_axis_name="s",
                                  num_cores=1)
    f = pl.kernel(
        _nms_body,
        out_type=jax.ShapeDtypeStruct((MAX_DET * 16,), jnp.float32),
        mesh=mesh,
        compiler_params=pltpu.CompilerParams(needs_layout_passes=False),
        scratch_types=[
            pltpu.VMEM((4 * NP,), jnp.float32),
            pltpu.VMEM((NP,), jnp.float32),
            pltpu.VMEM((NS1,), jnp.float32),
            pltpu.VMEM((NS2,), jnp.float32),
            pltpu.VMEM((5 * KPAD,), jnp.float32),
            pltpu.VMEM((MAX_DET * 16,), jnp.float32),
            pltpu.SemaphoreType.DMA,
        ],
    )
    return f(cat, sc)


def kernel(boxes, scores):
    out = _nms(boxes.reshape(-1), scores)
    return out.reshape(MAX_DET, 16)[:, :5]


# speedup vs baseline: 1.2482x; 1.2482x over previous
import functools

import jax
import jax.numpy as jnp
from jax import lax
from jax.experimental import pallas as pl
from jax.experimental.pallas import tpu as pltpu
from jax.experimental.pallas import tpu_sc as plsc

N = 20000
NP = 20480
NS1 = NP // 16
NS2 = NS1 // 16
KPAD = 112
MAX_DET = 100
SCORE_THRESH = 0.05
NMS_THRESH = 0.5
NEG_INF = float("-inf")


def _nms_body(cat_hbm, sc_hbm, out_hbm, cat_v, work_v, seg1_v, seg2_v,
              kcat_v, out_v, sem):
    wid = lax.axis_index("s")

    @pl.when(wid == 0)
    def _():
        iota = lax.broadcasted_iota(jnp.int32, (16,), 0)
        iota_f = iota.astype(jnp.float32)
        iota16 = iota * 16
        lane0 = iota == 0
        gplane = jnp.minimum(iota, 3) * NP
        neg16 = jnp.full((16,), NEG_INF, jnp.float32)

        cdma = pltpu.async_copy(cat_hbm, cat_v, sem)
        pltpu.sync_copy(sc_hbm, work_v)

        def s1(c, _):
            m = neg16
            for j in range(16):
                s = plsc.load_gather(work_v, [c * 256 + iota16 + j])
                m = jnp.maximum(m, jnp.where(s >= SCORE_THRESH, s, NEG_INF))
            seg1_v[pl.ds(c * 16, 16)] = m
            return 0
        lax.fori_loop(0, NS2, s1, 0, unroll=False)

        def s2(c, _):
            m = neg16
            for j in range(16):
                m = jnp.maximum(m, plsc.load_gather(seg1_v,
                                                    [c * 256 + iota16 + j]))
            seg2_v[pl.ds(c * 16, 16)] = m
            return 0
        lax.fori_loop(0, NS2 // 16, s2, 0, unroll=True)

        for p, val in enumerate((2.0, 2.0, -2.0, -2.0, 0.0)):
            for c in range(KPAD // 16):
                kcat_v[pl.ds(p * KPAD + c * 16, 16)] = jnp.full(
                    (16,), val, jnp.float32)

        cdma.wait()

        def pick():
            def seg_scan(c, carry):
                m, mi = carry
                v = seg2_v[pl.ds(c * 16, 16)]
                upd = v > m
                return jnp.where(upd, v, m), jnp.where(upd, c * 16.0 + iota_f,
                                                       mi)
            m, mi = lax.fori_loop(0, NS2 // 16, seg_scan, (neg16, iota_f),
                                  unroll=True)
            vmax = jnp.max(m)
            j2 = jnp.min(jnp.where(m == vmax, mi, jnp.float32(1e9)))
            j2 = j2.astype(jnp.int32)
            c1 = seg1_v[pl.ds(j2 * 16, 16)]
            j1 = j2 * 16 + jnp.minimum(plsc.all_reduce_ffs(c1 == vmax)[0], 15)
            c0 = work_v[pl.ds(j1 * 16, 16)]
            g = j1 * 16 + jnp.minimum(plsc.all_reduce_ffs(c0 == vmax)[0], 15)
            return vmax, g

        def kept_test(it, vmax, g):
            gc = plsc.load_gather(cat_v, [gplane + g])
            cx1, cy1, cx2, cy2 = gc[0], gc[1], gc[2], gc[3]
            carea = jnp.maximum(cx2 - cx1, 0.0) * jnp.maximum(cy2 - cy1, 0.0)
            nch = (it + 15) // 16

            def tb(k, acc):
                kx1 = kcat_v[pl.ds(k * 16, 16)]
                ky1 = kcat_v[pl.ds(KPAD + k * 16, 16)]
                kx2 = kcat_v[pl.ds(2 * KPAD + k * 16, 16)]
                ky2 = kcat_v[pl.ds(3 * KPAD + k * 16, 16)]
                karea = kcat_v[pl.ds(4 * KPAD + k * 16, 16)]
                x1 = jnp.maximum(kx1, cx1)
                y1 = jnp.maximum(ky1, cy1)
                x2 = jnp.minimum(kx2, cx2)
                y2 = jnp.minimum(ky2, cy2)
                inter = jnp.maximum(x2 - x1, 0.0) * jnp.maximum(y2 - y1, 0.0)
                union = karea + carea - inter
                iou = inter / jnp.maximum(union, 1e-9)
                return acc | (iou > NMS_THRESH)
            acc = lax.fori_loop(0, nch, tb, iota < 0, unroll=False)
            return jnp.any(acc) & (vmax > NEG_INF)

        def mark(g):
            plsc.store_scatter(work_v, [jnp.full((16,), g, jnp.int32)],
                               neg16, mask=lane0)
            j1 = g // 16
            c0 = work_v[pl.ds(j1 * 16, 16)]
            nm1 = jnp.max(jnp.where(c0 >= SCORE_THRESH, c0, NEG_INF))
            plsc.store_scatter(seg1_v, [jnp.full((16,), j1, jnp.int32)],
                               jnp.full((16,), nm1, jnp.float32), mask=lane0)
            j2 = j1 // 16
            nm2 = jnp.max(seg1_v[pl.ds(j2 * 16, 16)])
            plsc.store_scatter(seg2_v, [jnp.full((16,), j2, jnp.int32)],
                               jnp.full((16,), nm2, jnp.float32), mask=lane0)

        def round_body(it, _):
            vmax0, g0 = pick()
            fail0 = kept_test(it, vmax0, g0)

            def body(carry):
                _, g, _ = carry
                mark(g)
                vmax2, g2 = pick()
                return vmax2, g2, kept_test(it, vmax2, g2)
            vmax, g, _ = lax.while_loop(lambda c: c[2], body,
                                        (vmax0, g0, fail0))
            mark(g)
            valid = vmax > NEG_INF

            gc = plsc.load_gather(cat_v, [gplane + g])
            sent = jnp.where(iota < 2, 2.0, jnp.where(iota < 4, -2.0, 0.0))
            app = jnp.where(valid, gc, sent)
            ax1, ay1, ax2, ay2 = app[0], app[1], app[2], app[3]
            aarea = jnp.maximum(ax2 - ax1, 0.0) * jnp.maximum(ay2 - ay1, 0.0)
            for p, v in enumerate((ax1, ay1, ax2, ay2, aarea)):
                plsc.store_scatter(
                    kcat_v, [jnp.full((16,), p * KPAD, jnp.int32) + it],
                    jnp.full((16,), v, jnp.float32), mask=lane0)

            row = jnp.where(iota < 4, gc, jnp.where(iota == 4, vmax, 0.0))
            row = jnp.where(valid, row, jnp.zeros((16,), jnp.float32))
            out_v[pl.ds(it * 16, 16)] = row
            return 0

        lax.fori_loop(0, MAX_DET, round_body, 0, unroll=False)
        pltpu.sync_copy(out_v, out_hbm)


@jax.jit
def _nms(cat, sc):
    mesh = plsc.VectorSubcoreMesh(core_axis_name="c", subcore_axis_name="s",
                                  num_cores=1)
    f = pl.kernel(
        _nms_body,
        out_type=jax.ShapeDtypeStruct((MAX_DET * 16,), jnp.float32),
        mesh=mesh,
        compiler_params=pltpu.CompilerParams(needs_layout_passes=False),
        scratch_types=[
            pltpu.VMEM((4 * NP,), jnp.float32),
            pltpu.VMEM((NP,), jnp.float32),
            pltpu.VMEM((NS1,), jnp.float32),
            pltpu.VMEM((NS2,), jnp.float32),
            pltpu.VMEM((5 * KPAD,), jnp.float32),
            pltpu.VMEM((MAX_DET * 16,), jnp.float32),
            pltpu.SemaphoreType.DMA,
        ],
    )
    return f(cat, sc)


def kernel(boxes, scores):
    b = jnp.pad(boxes, ((0, NP - N), (0, 0)))
    s = jnp.pad(scores, ((0, NP - N),), constant_values=-1.0)
    cat = b.T.reshape(-1)
    out = _nms(cat, s)
    return out.reshape(MAX_DET, 16)[:, :5]


# speedup vs baseline: 1.3262x; 1.0625x over previous
import functools

import jax
import jax.numpy as jnp
from jax import lax
from jax.experimental import pallas as pl
from jax.experimental.pallas import tpu as pltpu
from jax.experimental.pallas import tpu_sc as plsc

N = 20000
NP = 20480
NS1 = NP // 16
NS2 = NS1 // 16
KPAD = 112
MAX_DET = 100
SCORE_THRESH = 0.05
NMS_THRESH = 0.5
NEG_INF = float("-inf")


def _nms_body(cat_hbm, sc_hbm, out_hbm, cat_v, work_v, seg1_v, seg2_v,
              kcat_v, out_v, sem):
    wid = lax.axis_index("s")

    @pl.when(wid == 0)
    def _():
        iota = lax.broadcasted_iota(jnp.int32, (16,), 0)
        iota_f = iota.astype(jnp.float32)
        iota16 = iota * 16
        lane0 = iota == 0
        gplane = jnp.minimum(iota, 3) * NP
        neg16 = jnp.full((16,), NEG_INF, jnp.float32)

        cdma = pltpu.async_copy(cat_hbm, cat_v, sem)
        pltpu.sync_copy(sc_hbm, work_v)

        def s1(c, _):
            m = neg16
            for j in range(16):
                s = plsc.load_gather(work_v, [c * 256 + iota16 + j])
                m = jnp.maximum(m, jnp.where(s >= SCORE_THRESH, s, NEG_INF))
            seg1_v[pl.ds(c * 16, 16)] = m
            return 0
        lax.fori_loop(0, NS2, s1, 0, unroll=False)

        def s2(c, _):
            m = neg16
            for j in range(16):
                m = jnp.maximum(m, plsc.load_gather(seg1_v,
                                                    [c * 256 + iota16 + j]))
            seg2_v[pl.ds(c * 16, 16)] = m
            return 0
        lax.fori_loop(0, NS2 // 16, s2, 0, unroll=True)

        for p, val in enumerate((2.0, 2.0, -2.0, -2.0, 0.0)):
            for c in range(KPAD // 16):
                kcat_v[pl.ds(p * KPAD + c * 16, 16)] = jnp.full(
                    (16,), val, jnp.float32)

        cdma.wait()

        def pick():
            def comb(a, b):
                (av, ai), (bv, bi) = a, b
                upd = bv > av
                return jnp.where(upd, bv, av), jnp.where(upd, bi, ai)
            ch = [(seg2_v[pl.ds(c * 16, 16)], c * 16.0 + iota_f)
                  for c in range(NS2 // 16)]
            m, mi = comb(comb(comb(ch[0], ch[1]), comb(ch[2], ch[3])), ch[4])
            vmax = jnp.max(m)
            j2 = jnp.min(jnp.where(m == vmax, mi, jnp.float32(1e9)))
            j2 = j2.astype(jnp.int32)
            c1 = seg1_v[pl.ds(j2 * 16, 16)]
            l1 = jnp.minimum(plsc.all_reduce_ffs(c1 == vmax)[0], 15)
            j1 = j2 * 16 + l1
            c0 = work_v[pl.ds(j1 * 16, 16)]
            l0 = jnp.minimum(plsc.all_reduce_ffs(c0 == vmax)[0], 15)
            g = j1 * 16 + l0
            return vmax, g, j2, j1, l1, l0, c1, c0

        def kept_test(vmax, g):
            gc = plsc.load_gather(cat_v, [gplane + g])
            cx1, cy1, cx2, cy2 = gc[0], gc[1], gc[2], gc[3]
            carea = jnp.maximum(cx2 - cx1, 0.0) * jnp.maximum(cy2 - cy1, 0.0)
            acc = iota < 0
            for k in range(KPAD // 16):
                kx1 = kcat_v[pl.ds(k * 16, 16)]
                ky1 = kcat_v[pl.ds(KPAD + k * 16, 16)]
                kx2 = kcat_v[pl.ds(2 * KPAD + k * 16, 16)]
                ky2 = kcat_v[pl.ds(3 * KPAD + k * 16, 16)]
                karea = kcat_v[pl.ds(4 * KPAD + k * 16, 16)]
                x1 = jnp.maximum(kx1, cx1)
                y1 = jnp.maximum(ky1, cy1)
                x2 = jnp.minimum(kx2, cx2)
                y2 = jnp.minimum(ky2, cy2)
                inter = jnp.maximum(x2 - x1, 0.0) * jnp.maximum(y2 - y1, 0.0)
                union = karea + carea - inter
                iou = inter / jnp.maximum(union, 1e-9)
                acc = acc | (iou > NMS_THRESH)
            return jnp.any(acc) & (vmax > NEG_INF)

        def mark(g, j2, j1, l1, l0, c1, c0):
            plsc.store_scatter(work_v, [jnp.full((16,), g, jnp.int32)],
                               neg16, mask=lane0)
            c0n = jnp.where(iota == l0, NEG_INF, c0)
            nm1 = jnp.max(jnp.where(c0n >= SCORE_THRESH, c0n, NEG_INF))
            plsc.store_scatter(seg1_v, [jnp.full((16,), j1, jnp.int32)],
                               jnp.full((16,), nm1, jnp.float32), mask=lane0)
            c1n = jnp.where(iota == l1, nm1, c1)
            nm2 = jnp.max(c1n)
            plsc.store_scatter(seg2_v, [jnp.full((16,), j2, jnp.int32)],
                               jnp.full((16,), nm2, jnp.float32), mask=lane0)

        def round_body(it, _):
            st0 = pick()
            fail0 = kept_test(st0[0], st0[1])

            def body(carry):
                st, _ = carry
                mark(*st[1:])
                st2 = pick()
                return st2, kept_test(st2[0], st2[1])
            st, _ = lax.while_loop(lambda c: c[1], body, (st0, fail0))
            vmax, g = st[0], st[1]
            mark(*st[1:])
            valid = vmax > NEG_INF

            gc = plsc.load_gather(cat_v, [gplane + g])
            sent = jnp.where(iota < 2, 2.0, jnp.where(iota < 4, -2.0, 0.0))
            app = jnp.where(valid, gc, sent)
            ax1, ay1, ax2, ay2 = app[0], app[1], app[2], app[3]
            aarea = jnp.maximum(ax2 - ax1, 0.0) * jnp.maximum(ay2 - ay1, 0.0)
            for p, v in enumerate((ax1, ay1, ax2, ay2, aarea)):
                plsc.store_scatter(
                    kcat_v, [jnp.full((16,), p * KPAD, jnp.int32) + it],
                    jnp.full((16,), v, jnp.float32), mask=lane0)

            row = jnp.where(iota < 4, gc, jnp.where(iota == 4, vmax, 0.0))
            row = jnp.where(valid, row, jnp.zeros((16,), jnp.float32))
            out_v[pl.ds(it * 16, 16)] = row
            return 0

        lax.fori_loop(0, MAX_DET, round_body, 0, unroll=False)
        pltpu.sync_copy(out_v, out_hbm)


@jax.jit
def _nms(cat, sc):
    mesh = plsc.VectorSubcoreMesh(core_axis_name="c", subcore_axis_name="s",
                                  num_cores=1)
    f = pl.kernel(
        _nms_body,
        out_type=jax.ShapeDtypeStruct((MAX_DET * 16,), jnp.float32),
        mesh=mesh,
        compiler_params=pltpu.CompilerParams(needs_layout_passes=False),
        scratch_types=[
            pltpu.VMEM((4 * NP,), jnp.float32),
            pltpu.VMEM((NP,), jnp.float32),
            pltpu.VMEM((NS1,), jnp.float32),
            pltpu.VMEM((NS2,), jnp.float32),
            pltpu.VMEM((5 * KPAD,), jnp.float32),
            pltpu.VMEM((MAX_DET * 16,), jnp.float32),
            pltpu.SemaphoreType.DMA,
        ],
    )
    return f(cat, sc)


def kernel(boxes, scores):
    b = jnp.pad(boxes, ((0, NP - N), (0, 0)))
    s = jnp.pad(scores, ((0, NP - N),), constant_values=-1.0)
    cat = b.T.reshape(-1)
    out = _nms(cat, s)
    return out.reshape(MAX_DET, 16)[:, :5]


# speedup vs baseline: 1.3704x; 1.0333x over previous
import functools

import jax
import jax.numpy as jnp
from jax import lax
from jax.experimental import pallas as pl
from jax.experimental.pallas import tpu as pltpu
from jax.experimental.pallas import tpu_sc as plsc

N = 20000
NP = 20480
NS1 = NP // 16
NC2 = NS1 // 256
KPAD = 112
MAX_DET = 100
SCORE_THRESH = 0.05
NMS_THRESH = 0.5
ULP = 2.0 ** -24
NEG_INF = float("-inf")


def _nms_body(cat_hbm, sc_hbm, out_hbm, cat_v, work_v, seg1_v, kcat_v,
              out_v, sem):
    wid = lax.axis_index("s")

    @pl.when(wid == 0)
    def _():
        iota = lax.broadcasted_iota(jnp.int32, (16,), 0)
        iota_f = iota.astype(jnp.float32)
        iota16 = iota * 16
        lane0 = iota == 0
        gplane = jnp.minimum(iota, 3) * NP
        neg16 = jnp.full((16,), NEG_INF, jnp.float32)

        cdma = pltpu.async_copy(cat_hbm, cat_v, sem)
        pltpu.sync_copy(sc_hbm, work_v)

        def s1(c, _):
            m = neg16
            for j in range(16):
                s = plsc.load_gather(work_v, [c * 256 + iota16 + j])
                m = jnp.maximum(m, jnp.where(s >= SCORE_THRESH, s, NEG_INF))
            seg1_v[pl.ds(c * 16, 16)] = m
            return 0
        lax.fori_loop(0, NS1 // 16, s1, 0, unroll=False)

        ch0 = []
        for c in range(NC2):
            m = neg16
            for j in range(16):
                m = jnp.maximum(m, plsc.load_gather(seg1_v,
                                                    [c * 256 + iota16 + j]))
            ch0.append(m)
        ch0 = tuple(ch0)

        for p, val in enumerate((2.0, 2.0, -2.0, -2.0, 0.0)):
            for c in range(KPAD // 16):
                kcat_v[pl.ds(p * KPAD + c * 16, 16)] = jnp.full(
                    (16,), val, jnp.float32)

        cdma.wait()

        def pick(ch):
            def comb(a, b):
                (av, ai), (bv, bi) = a, b
                upd = bv > av
                return jnp.where(upd, bv, av), jnp.where(upd, bi, ai)
            pairs = [(ch[c], c * 16.0 + iota_f) for c in range(NC2)]
            m, mi = comb(comb(comb(pairs[0], pairs[1]),
                              comb(pairs[2], pairs[3])), pairs[4])
            vmax = jnp.max(m)
            j2 = jnp.min(jnp.where(m == vmax, mi, jnp.float32(1e9)))
            j2 = j2.astype(jnp.int32)
            c1 = seg1_v[pl.ds(j2 * 16, 16)]
            l1 = jnp.minimum(plsc.all_reduce_ffs(c1 == vmax)[0], 15)
            j1 = j2 * 16 + l1
            c0 = work_v[pl.ds(j1 * 16, 16)]
            l0 = jnp.minimum(plsc.all_reduce_ffs(c0 == vmax)[0], 15)
            g = j1 * 16 + l0
            return vmax, g, j2, j1, l1, l0, c1, c0

        def kept_test(vmax, g):
            gc = plsc.load_gather(cat_v, [gplane + g])
            cx1, cy1, cx2, cy2 = gc[0], gc[1], gc[2], gc[3]
            carea = jnp.maximum(cx2 - cx1, 0.0) * jnp.maximum(cy2 - cy1, 0.0)
            acc = iota < 0
            for k in range(KPAD // 16):
                kx1 = kcat_v[pl.ds(k * 16, 16)]
                ky1 = kcat_v[pl.ds(KPAD + k * 16, 16)]
                kx2 = kcat_v[pl.ds(2 * KPAD + k * 16, 16)]
                ky2 = kcat_v[pl.ds(3 * KPAD + k * 16, 16)]
                karea = kcat_v[pl.ds(4 * KPAD + k * 16, 16)]
                x1 = jnp.maximum(kx1, cx1)
                y1 = jnp.maximum(ky1, cy1)
                x2 = jnp.minimum(kx2, cx2)
                y2 = jnp.minimum(ky2, cy2)
                inter = jnp.maximum(x2 - x1, 0.0) * jnp.maximum(y2 - y1, 0.0)
                union = jnp.maximum(karea + carea - inter, 1e-9)
                acc = acc | (inter + inter - union > union * ULP)
            return (plsc.all_reduce_population_count(acc)[0] > 0) & \
                (vmax > NEG_INF)

        def mark(ch, g, j2, j1, l1, l0, c1, c0):
            plsc.store_scatter(work_v, [jnp.full((16,), g, jnp.int32)],
                               neg16, mask=lane0)
            c0n = jnp.where(iota == l0, NEG_INF, c0)
            nm1 = jnp.max(jnp.where(c0n >= SCORE_THRESH, c0n, NEG_INF))
            plsc.store_scatter(seg1_v, [jnp.full((16,), j1, jnp.int32)],
                               jnp.full((16,), nm1, jnp.float32), mask=lane0)
            nm2 = jnp.max(jnp.where(iota == l1, nm1, c1))
            return tuple(jnp.where((c * 16 + iota) == j2, nm2, ch[c])
                         for c in range(NC2))

        def round_body(it, ch):
            st0 = pick(ch)
            fail0 = kept_test(st0[0], st0[1])

            def body(carry):
                ch_, st, _ = carry
                ch_ = mark(ch_, *st[1:])
                st2 = pick(ch_)
                return ch_, st2, kept_test(st2[0], st2[1])
            ch, st, _ = lax.while_loop(lambda c: c[2], body,
                                       (ch, st0, fail0))
            vmax, g = st[0], st[1]
            ch = mark(ch, *st[1:])
            valid = vmax > NEG_INF

            gc = plsc.load_gather(cat_v, [gplane + g])
            sent = jnp.where(iota < 2, 2.0, jnp.where(iota < 4, -2.0, 0.0))
            app = jnp.where(valid, gc, sent)
            ax1, ay1, ax2, ay2 = app[0], app[1], app[2], app[3]
            aarea = jnp.maximum(ax2 - ax1, 0.0) * jnp.maximum(ay2 - ay1, 0.0)
            for p, v in enumerate((ax1, ay1, ax2, ay2, aarea)):
                plsc.store_scatter(
                    kcat_v, [jnp.full((16,), p * KPAD, jnp.int32) + it],
                    jnp.full((16,), v, jnp.float32), mask=lane0)

            row = jnp.where(iota < 4, gc, jnp.where(iota == 4, vmax, 0.0))
            row = jnp.where(valid, row, jnp.zeros((16,), jnp.float32))
            out_v[pl.ds(it * 16, 16)] = row
            return ch

        lax.fori_loop(0, MAX_DET, round_body, ch0, unroll=False)
        pltpu.sync_copy(out_v, out_hbm)


@jax.jit
def _nms(cat, sc):
    mesh = plsc.VectorSubcoreMesh(core_axis_name="c", subcore_axis_name="s",
                                  num_cores=1)
    f = pl.kernel(
        _nms_body,
        out_type=jax.ShapeDtypeStruct((MAX_DET * 16,), jnp.float32),
        mesh=mesh,
        compiler_params=pltpu.CompilerParams(needs_layout_passes=False),
        scratch_types=[
            pltpu.VMEM((4 * NP,), jnp.float32),
            pltpu.VMEM((NP,), jnp.float32),
            pltpu.VMEM((NS1,), jnp.float32),
            pltpu.VMEM((5 * KPAD,), jnp.float32),
            pltpu.VMEM((MAX_DET * 16,), jnp.float32),
            pltpu.SemaphoreType.DMA,
        ],
    )
    return f(cat, sc)


def kernel(boxes, scores):
    b = jnp.pad(boxes, ((0, NP - N), (0, 0)))
    s = jnp.pad(scores, ((0, NP - N),), constant_values=-1.0)
    cat = b.T.reshape(-1)
    out = _nms(cat, s)
    return out.reshape(MAX_DET, 16)[:, :5]


# speedup vs baseline: 1.4053x; 1.0255x over previous
import functools

import jax
import jax.numpy as jnp
from jax import lax
from jax.experimental import pallas as pl
from jax.experimental.pallas import tpu as pltpu
from jax.experimental.pallas import tpu_sc as plsc

N = 20000
NP = 20480
NS1 = NP // 16
NC2 = NS1 // 256
KPAD = 112
MAX_DET = 100
SCORE_THRESH = 0.05
NMS_THRESH = 0.5
ULP = 2.0 ** -24
NEG_INF = float("-inf")


def _nms_body(cat_hbm, sc_hbm, out_hbm, cat_v, work_v, seg1_v, kcat_v,
              out_v, sem):
    wid = lax.axis_index("s")

    @pl.when(wid == 0)
    def _():
        iota = lax.broadcasted_iota(jnp.int32, (16,), 0)
        iota_f = iota.astype(jnp.float32)
        iota16 = iota * 16
        lane0 = iota == 0
        gplane = jnp.minimum(iota, 3) * NP
        neg16 = jnp.full((16,), NEG_INF, jnp.float32)

        pltpu.sync_copy(sc_hbm, work_v)
        cdma = pltpu.async_copy(cat_hbm, cat_v, sem)

        def s1(c, _):
            m = neg16
            for j in range(16):
                s = plsc.load_gather(work_v, [c * 256 + iota16 + j])
                m = jnp.maximum(m, jnp.where(s >= SCORE_THRESH, s, NEG_INF))
            seg1_v[pl.ds(c * 16, 16)] = m
            return 0
        lax.fori_loop(0, NS1 // 16, s1, 0, unroll=4)

        ch0 = []
        for c in range(NC2):
            m = neg16
            for j in range(16):
                m = jnp.maximum(m, plsc.load_gather(seg1_v,
                                                    [c * 256 + iota16 + j]))
            ch0.append(m)
        ch0 = tuple(ch0)

        for p, val in enumerate((2.0, 2.0, -2.0, -2.0)):
            for c in range(KPAD // 16):
                kcat_v[pl.ds(p * KPAD + c * 16, 16)] = jnp.full(
                    (16,), val, jnp.float32)

        cdma.wait()

        def pick(ch):
            m = jnp.maximum(jnp.maximum(jnp.maximum(ch[0], ch[1]),
                                        jnp.maximum(ch[2], ch[3])), ch[4])
            vmax = jnp.max(m)
            ms = [ch[c] == vmax for c in range(NC2)]
            ps = [plsc.all_reduce_population_count(ms[c])[0] > 0
                  for c in range(NC2)]
            cstar = jnp.where(
                ps[0], 0, jnp.where(ps[1], 1, jnp.where(
                    ps[2], 2, jnp.where(ps[3], 3, 4))))
            mstar = jnp.where(
                ps[0], ms[0], jnp.where(ps[1], ms[1], jnp.where(
                    ps[2], ms[2], jnp.where(ps[3], ms[3], ms[4]))))
            lane2 = jnp.minimum(plsc.all_reduce_ffs(mstar)[0], 15)
            j2 = cstar * 16 + lane2
            c1 = seg1_v[pl.ds(j2 * 16, 16)]
            l1 = jnp.minimum(plsc.all_reduce_ffs(c1 == vmax)[0], 15)
            j1 = j2 * 16 + l1
            c0 = work_v[pl.ds(j1 * 16, 16)]
            l0 = jnp.minimum(plsc.all_reduce_ffs(c0 == vmax)[0], 15)
            g = j1 * 16 + l0
            return vmax, g, j2, j1, l1, l0, c1, c0

        def kept_test(vmax, g):
            gc = plsc.load_gather(cat_v, [gplane + g])
            cx1, cy1, cx2, cy2 = gc[0], gc[1], gc[2], gc[3]
            carea = jnp.maximum(cx2 - cx1, 0.0) * jnp.maximum(cy2 - cy1, 0.0)
            acc = iota < 0
            for k in range(KPAD // 16):
                kx1 = kcat_v[pl.ds(k * 16, 16)]
                ky1 = kcat_v[pl.ds(KPAD + k * 16, 16)]
                kx2 = kcat_v[pl.ds(2 * KPAD + k * 16, 16)]
                ky2 = kcat_v[pl.ds(3 * KPAD + k * 16, 16)]
                karea = jnp.maximum(kx2 - kx1, 0.0) * \
                    jnp.maximum(ky2 - ky1, 0.0)
                x1 = jnp.maximum(kx1, cx1)
                y1 = jnp.maximum(ky1, cy1)
                x2 = jnp.minimum(kx2, cx2)
                y2 = jnp.minimum(ky2, cy2)
                inter = jnp.maximum(x2 - x1, 0.0) * jnp.maximum(y2 - y1, 0.0)
                union = jnp.maximum(karea + carea - inter, 1e-9)
                acc = acc | (inter + inter - union > union * ULP)
            return (plsc.all_reduce_population_count(acc)[0] > 0) & \
                (vmax > NEG_INF)

        def mark(ch, g, j2, j1, l1, l0, c1, c0):
            plsc.store_scatter(work_v, [jnp.full((16,), g, jnp.int32)],
                               neg16, mask=lane0)
            c0n = jnp.where(iota == l0, NEG_INF, c0)
            nm1 = jnp.max(jnp.where(c0n >= SCORE_THRESH, c0n, NEG_INF))
            plsc.store_scatter(seg1_v, [jnp.full((16,), j1, jnp.int32)],
                               jnp.full((16,), nm1, jnp.float32), mask=lane0)
            nm2 = jnp.max(jnp.where(iota == l1, nm1, c1))
            return tuple(jnp.where((c * 16 + iota) == j2, nm2, ch[c])
                         for c in range(NC2))

        def round_body(it, ch):
            st0 = pick(ch)
            fail0 = kept_test(st0[0], st0[1])

            def body(carry):
                ch_, st, _ = carry
                ch_ = mark(ch_, *st[1:])
                st2 = pick(ch_)
                return ch_, st2, kept_test(st2[0], st2[1])
            ch, st, _ = lax.while_loop(lambda c: c[2], body,
                                       (ch, st0, fail0))
            vmax, g = st[0], st[1]
            ch = mark(ch, *st[1:])
            valid = vmax > NEG_INF

            gc = plsc.load_gather(cat_v, [gplane + g])
            sent = jnp.where(iota < 2, 2.0, jnp.where(iota < 4, -2.0, 0.0))
            app = jnp.where(valid, gc, sent)
            ax1, ay1, ax2, ay2 = app[0], app[1], app[2], app[3]
            for p, v in enumerate((ax1, ay1, ax2, ay2)):
                plsc.store_scatter(
                    kcat_v, [jnp.full((16,), p * KPAD, jnp.int32) + it],
                    jnp.full((16,), v, jnp.float32), mask=lane0)

            row = jnp.where(iota < 4, gc, jnp.where(iota == 4, vmax, 0.0))
            row = jnp.where(valid, row, jnp.zeros((16,), jnp.float32))
            out_v[pl.ds(it * 16, 16)] = row
            return ch

        lax.fori_loop(0, MAX_DET, round_body, ch0, unroll=False)
        pltpu.sync_copy(out_v, out_hbm)


@jax.jit
def _nms(cat, sc):
    mesh = plsc.VectorSubcoreMesh(core_axis_name="c", subcore_axis_name="s",
                                  num_cores=1)
    f = pl.kernel(
        _nms_body,
        out_type=jax.ShapeDtypeStruct((MAX_DET * 16,), jnp.float32),
        mesh=mesh,
        compiler_params=pltpu.CompilerParams(needs_layout_passes=False),
        scratch_types=[
            pltpu.VMEM((4 * NP,), jnp.float32),
            pltpu.VMEM((NP,), jnp.float32),
            pltpu.VMEM((NS1,), jnp.float32),
            pltpu.VMEM((4 * KPAD,), jnp.float32),
            pltpu.VMEM((MAX_DET * 16,), jnp.float32),
            pltpu.SemaphoreType.DMA,
        ],
    )
    return f(cat, sc)


def kernel(boxes, scores):
    b = jnp.pad(boxes, ((0, NP - N), (0, 0)))
    s = jnp.pad(scores, ((0, NP - N),), constant_values=-1.0)
    cat = b.T.reshape(-1)
    out = _nms(cat, s)
    return out.reshape(MAX_DET, 16)[:, :5]
